# Initial kernel scaffold; baseline (speedup 1.0000x reference)
import functools

import jax
import jax.numpy as jnp
from jax import lax
from jax.experimental import pallas as pl
from jax.experimental.pallas import tpu as pltpu
from jax.experimental.pallas import tpu_sc as plsc

N = 10000
E = 160000
D = 256
H = 8
DH = 32
HALF = 128
NS = 16
EPT = E // NS
CH = 80
NCHUNK = EPT // CH
NPT = N // NS
F32 = jnp.float32


def _proj3_body(x_ref, wq_ref, wk_ref, wv_ref, q_ref, k_ref, v_ref):
    x = x_ref[...]
    scale = jnp.float32(1.0 / (DH ** 0.5))
    q_ref[0] = jnp.dot(x, wq_ref[...], preferred_element_type=F32) * scale
    k_ref[0] = jnp.dot(x, wk_ref[...], preferred_element_type=F32)
    v_ref[0] = jnp.dot(x, wv_ref[...], preferred_element_type=F32)


def _proj3(x, wq, wk, wv, bn):
    n = x.shape[0]
    wspec = pl.BlockSpec((D, HALF), lambda c, i: (0, c))
    ospec = pl.BlockSpec((1, bn, HALF), lambda c, i: (c, i, 0))
    oshape = jax.ShapeDtypeStruct((2, n, HALF), F32)
    return pl.pallas_call(
        _proj3_body,
        grid=(2, n // bn),
        in_specs=[pl.BlockSpec((bn, D), lambda c, i: (i, 0)), wspec, wspec, wspec],
        out_specs=[ospec, ospec, ospec],
        out_shape=[oshape, oshape, oshape],
    )(x, wq, wk, wv)


def _proj1_body(x_ref, w_ref, o_ref):
    o_ref[0] = jnp.dot(x_ref[...], w_ref[...], preferred_element_type=F32)


def _proj1(x, w, bn):
    n = x.shape[0]
    return pl.pallas_call(
        _proj1_body,
        grid=(2, n // bn),
        in_specs=[pl.BlockSpec((bn, D), lambda c, i: (i, 0)),
                  pl.BlockSpec((D, HALF), lambda c, i: (0, c))],
        out_specs=pl.BlockSpec((1, bn, HALF), lambda c, i: (c, i, 0)),
        out_shape=jax.ShapeDtypeStruct((2, n, HALF), F32),
    )(x, w)


def _sc_attention_body(ei, qt, kt, vt, pet, eattn_o, hattn_o,
                       src_i, dst_i, dst_g, kr, qr, vr, pr, sr, wr, zr,
                       zbuf, zzbuf, wv_acc, z_acc, sem0, sem1, sem2):
    c = lax.axis_index("c")
    s = lax.axis_index("s")
    cN = c * N
    cE = c * E
    zv = jnp.zeros((16,), F32)

    def _z_zbuf(i, _):
        for k in range(HALF // 16):
            zbuf[i, pl.ds(k * 16, 16)] = zv
        return 0
    lax.fori_loop(0, 125, _z_zbuf, 0)

    def _z_zzbuf(i, _):
        zzbuf[i, pl.ds(0, 16)] = zv
        return 0
    lax.fori_loop(0, NPT, _z_zzbuf, 0)

    def _z_zr(i, _):
        zr[i, pl.ds(0, 16)] = zv
        return 0
    lax.fori_loop(0, CH, _z_zr, 0)

    for j in range(NPT // 125):
        pltpu.sync_copy(zbuf, wv_acc.at[pl.ds(s * NPT + j * 125, 125)])
    pltpu.sync_copy(zzbuf, z_acc.at[pl.ds(s * NPT, NPT)])
    plsc.subcore_barrier()

    def _chunk(ci, _):
        base = s * EPT + ci * CH
        pltpu.sync_copy(ei.at[0, pl.ds(base, CH)], src_i)
        pltpu.sync_copy(ei.at[1, pl.ds(base, CH)], dst_i)

        def _offs(g, _):
            src_i[pl.ds(g * 16, 16)] = src_i[pl.ds(g * 16, 16)] + cN
            dst_g[pl.ds(g * 16, 16)] = dst_i[pl.ds(g * 16, 16)] + cN
            return 0
        lax.fori_loop(0, CH // 16, _offs, 0)

        d0 = pltpu.async_copy(kt.at[src_i], kr, sem0)
        d1 = pltpu.async_copy(qt.at[dst_g], qr, sem1)
        d2 = pltpu.async_copy(vt.at[src_i], vr, sem2)
        pltpu.sync_copy(pet.at[pl.ds(cE + base, CH)], pr)
        d0.wait()
        d1.wait()
        d2.wait()

        for g in range(CH // 16):
            rows = g * 16 + lax.iota(jnp.int32, 16)
            for hh in range(4):
                def _dims(d2i, acc, _hh=hh, _rows=rows):
                    col = jnp.full((16,), _hh * DH, jnp.int32) + d2i
                    kv = plsc.load_gather(kr, [_rows, col])
                    qv = plsc.load_gather(qr, [_rows, col])
                    pv = plsc.load_gather(pr, [_rows, col])
                    sv = kv * qv * pv
                    plsc.store_scatter(sr, [_rows, col], sv)
                    return acc + sv
                acc = lax.fori_loop(0, DH, _dims, zv)
                sexp = jnp.exp(jnp.clip(acc, -5.0, 5.0))
                plsc.store_scatter(zr, [rows, jnp.full((16,), hh, jnp.int32)], sexp)

                def _dims2(d2i, carry, _hh=hh, _rows=rows, _sexp=sexp):
                    col = jnp.full((16,), _hh * DH, jnp.int32) + d2i
                    vv = plsc.load_gather(vr, [_rows, col])
                    plsc.store_scatter(wr, [_rows, col], vv * _sexp)
                    return carry
                lax.fori_loop(0, DH, _dims2, 0)

        pltpu.sync_copy(sr, eattn_o.at[pl.ds(cE + base, CH)])
        pltpu.sync_copy(wr, wv_acc.at[dst_i], add=True)
        pltpu.sync_copy(zr, z_acc.at[dst_i], add=True)
        return 0
    lax.fori_loop(0, NCHUNK, _chunk, 0)
    plsc.subcore_barrier()

    eps = jnp.float32(1e-6)
    for j in range(NPT // 125):
        r0 = s * NPT + j * 125
        pltpu.sync_copy(wv_acc.at[pl.ds(r0, 125)], zbuf)
        pltpu.sync_copy(z_acc.at[pl.ds(r0, 125)], zzbuf.at[pl.ds(0, 125)])

        def _div(r, _):
            for head in range(4):
                zb = plsc.load_gather(
                    zzbuf, [jnp.full((16,), r, jnp.int32),
                            jnp.full((16,), head, jnp.int32)])
                rec = jnp.float32(1.0) / (zb + eps)
                for k in (2 * head, 2 * head + 1):
                    zbuf[r, pl.ds(k * 16, 16)] = zbuf[r, pl.ds(k * 16, 16)] * rec
            return 0
        lax.fori_loop(0, 125, _div, 0)
        pltpu.sync_copy(zbuf, hattn_o.at[pl.ds(cN + r0, 125)])


def _sc_attention(edge_index, qt, kt, vt, pet):
    mesh = plsc.VectorSubcoreMesh(core_axis_name="c", subcore_axis_name="s")
    kern = pl.kernel(
        _sc_attention_body,
        out_type=[
            jax.ShapeDtypeStruct((2 * E, HALF), F32),
            jax.ShapeDtypeStruct((2 * N, HALF), F32),
        ],
        mesh=mesh,
        scratch_types=[
            pltpu.VMEM((CH,), jnp.int32),
            pltpu.VMEM((CH,), jnp.int32),
            pltpu.VMEM((CH,), jnp.int32),
            pltpu.VMEM((CH, HALF), F32),
            pltpu.VMEM((CH, HALF), F32),
            pltpu.VMEM((CH, HALF), F32),
            pltpu.VMEM((CH, HALF), F32),
            pltpu.VMEM((CH, HALF), F32),
            pltpu.VMEM((CH, HALF), F32),
            pltpu.VMEM((CH, 16), F32),
            pltpu.VMEM((125, HALF), F32),
            pltpu.VMEM((NPT, 16), F32),
            pltpu.VMEM_SHARED((N, HALF), F32),
            pltpu.VMEM_SHARED((N, 16), F32),
            pltpu.SemaphoreType.DMA,
            pltpu.SemaphoreType.DMA,
            pltpu.SemaphoreType.DMA,
        ],
    )
    return kern(edge_index, qt, kt, vt, pet)


def _postA_body(x_ref, aA_ref, aB_ref, wA_ref, wB_ref, b_ref,
                out_ref, s_ref, q_ref):
    i = pl.program_id(0)
    acc = (x_ref[...]
           + jnp.dot(aA_ref[0], wA_ref[...], preferred_element_type=F32)
           + jnp.dot(aB_ref[0], wB_ref[...], preferred_element_type=F32)
           + b_ref[...])
    out_ref[...] = acc

    @pl.when(i == 0)
    def _():
        s_ref[...] = jnp.zeros_like(s_ref)
        q_ref[...] = jnp.zeros_like(q_ref)

    s_ref[...] += jnp.sum(acc, axis=0, keepdims=True)
    q_ref[...] += jnp.sum(acc * acc, axis=0, keepdims=True)


def _postA(x, attn2, wA, wB, b, bn):
    n = x.shape[0]
    sspec = pl.BlockSpec((1, D), lambda i: (0, 0))
    return pl.pallas_call(
        _postA_body,
        grid=(n // bn,),
        in_specs=[pl.BlockSpec((bn, D), lambda i: (i, 0)),
                  pl.BlockSpec((1, bn, HALF), lambda i: (0, i, 0)),
                  pl.BlockSpec((1, bn, HALF), lambda i: (1, i, 0)),
                  pl.BlockSpec((HALF, D), lambda i: (0, 0)),
                  pl.BlockSpec((HALF, D), lambda i: (0, 0)),
                  pl.BlockSpec((1, D), lambda i: (0, 0))],
        out_specs=[pl.BlockSpec((bn, D), lambda i: (i, 0)), sspec, sspec],
        out_shape=[jax.ShapeDtypeStruct((n, D), F32),
                   jax.ShapeDtypeStruct((1, D), F32),
                   jax.ShapeDtypeStruct((1, D), F32)],
    )(x, attn2, attn2, wA, wB, b)


def _postB_body(x_ref, a1_ref, c1_ref, w1_ref, b1_ref, w2_ref, b2_ref,
                out_ref, s_ref, q_ref):
    i = pl.program_id(0)
    x = x_ref[...] * a1_ref[...] + c1_ref[...]
    t = jnp.maximum(jnp.dot(x, w1_ref[...], preferred_element_type=F32)
                    + b1_ref[...], 0.0)
    y = x + jnp.dot(t, w2_ref[...], preferred_element_type=F32) + b2_ref[...]
    out_ref[...] = y

    @pl.when(i == 0)
    def _():
        s_ref[...] = jnp.zeros_like(s_ref)
        q_ref[...] = jnp.zeros_like(q_ref)

    s_ref[...] += jnp.sum(y, axis=0, keepdims=True)
    q_ref[...] += jnp.sum(y * y, axis=0, keepdims=True)


def _postB(x, a1, c1, w1, b1, w2, b2, bn):
    n = x.shape[0]
    sspec = pl.BlockSpec((1, D), lambda i: (0, 0))
    return pl.pallas_call(
        _postB_body,
        grid=(n // bn,),
        in_specs=[pl.BlockSpec((bn, D), lambda i: (i, 0)),
                  pl.BlockSpec((1, D), lambda i: (0, 0)),
                  pl.BlockSpec((1, D), lambda i: (0, 0)),
                  pl.BlockSpec((D, 2 * D), lambda i: (0, 0)),
                  pl.BlockSpec((1, 2 * D), lambda i: (0, 0)),
                  pl.BlockSpec((2 * D, D), lambda i: (0, 0)),
                  pl.BlockSpec((1, D), lambda i: (0, 0))],
        out_specs=[pl.BlockSpec((bn, D), lambda i: (i, 0)), sspec, sspec],
        out_shape=[jax.ShapeDtypeStruct((n, D), F32),
                   jax.ShapeDtypeStruct((1, D), F32),
                   jax.ShapeDtypeStruct((1, D), F32)],
    )(x, a1, c1, w1, b1, w2, b2)


def _postC_body(x_ref, a_ref, c_ref, out_ref):
    out_ref[...] = x_ref[...] * a_ref[...] + c_ref[...]


def _postC(x, a, c, bn):
    n = x.shape[0]
    return pl.pallas_call(
        _postC_body,
        grid=(n // bn,),
        in_specs=[pl.BlockSpec((bn, D), lambda i: (i, 0)),
                  pl.BlockSpec((1, D), lambda i: (0, 0)),
                  pl.BlockSpec((1, D), lambda i: (0, 0))],
        out_specs=pl.BlockSpec((bn, D), lambda i: (i, 0)),
        out_shape=jax.ShapeDtypeStruct((n, D), F32),
    )(x, a, c)


def _bn_coef(ssum, sqsum, n, g, b):
    mu = ssum[0] / n
    var = sqsum[0] / n - mu * mu
    a = g / jnp.sqrt(var + 1e-5)
    return a.reshape(1, D), (b - mu * a).reshape(1, D)


def kernel(h, e, edge_index, WQ, WK, WV, We, WOh, bOh, WOe, bOe,
           W1h, b1h, W2h, b2h, W1e, b1e, W2e, b2e,
           g1h, be1h, g1e, be1e, g2h, be2h, g2e, be2e):
    q2, k2, v2 = _proj3(h, WQ, WK, WV, bn=2000)
    pe2 = _proj1(e, We, bn=2000)

    eattn, hattn = _sc_attention(
        edge_index,
        q2.reshape(2 * N, HALF), k2.reshape(2 * N, HALF),
        v2.reshape(2 * N, HALF), pe2.reshape(2 * E, HALF))

    hattn2 = hattn.reshape(2, N, HALF)
    eattn2 = eattn.reshape(2, E, HALF)

    hh, hs1, hq1 = _postA(h, hattn2, WOh[:HALF], WOh[HALF:],
                          bOh.reshape(1, D), bn=2000)
    ee, es1, eq1 = _postA(e, eattn2, WOe[:HALF], WOe[HALF:],
                          bOe.reshape(1, D), bn=2000)

    ha1, hc1 = _bn_coef(hs1, hq1, N, g1h, be1h)
    ea1, ec1 = _bn_coef(es1, eq1, E, g1e, be1e)

    hy, hs2, hq2 = _postB(hh, ha1, hc1, W1h, b1h.reshape(1, 2 * D),
                          W2h, b2h.reshape(1, D), bn=2000)
    ey, es2, eq2 = _postB(ee, ea1, ec1, W1e, b1e.reshape(1, 2 * D),
                          W2e, b2e.reshape(1, D), bn=2000)

    ha2, hc2 = _bn_coef(hs2, hq2, N, g2h, be2h)
    ea2, ec2 = _bn_coef(es2, eq2, E, g2e, be2e)

    h_out = _postC(hy, ha2, hc2, bn=2000)
    e_out = _postC(ey, ea2, ec2, bn=2000)
    return (h_out, e_out)


# speedup vs baseline: 3.6752x; 3.6752x over previous
import functools

import jax
import jax.numpy as jnp
from jax import lax
from jax.experimental import pallas as pl
from jax.experimental.pallas import tpu as pltpu
from jax.experimental.pallas import tpu_sc as plsc

N = 10000
E = 160000
D = 256
H = 8
DH = 32
HALF = 128
NS = 16
EPT = E // NS
CH = 80
NCHUNK = EPT // CH
NPAD = 10240
NPT = NPAD // NS
F32 = jnp.float32


def _proj3_body(x_ref, wq_ref, wk_ref, wv_ref, q_ref, k_ref, v_ref):
    x = x_ref[...]
    scale = jnp.float32(1.0 / (DH ** 0.5))
    q_ref[0] = jnp.dot(x, wq_ref[...], preferred_element_type=F32) * scale
    k_ref[0] = jnp.dot(x, wk_ref[...], preferred_element_type=F32)
    v_ref[0] = jnp.dot(x, wv_ref[...], preferred_element_type=F32)


def _proj3(x, wq, wk, wv, bn):
    n = x.shape[0]
    wspec = pl.BlockSpec((D, HALF), lambda c, i: (0, c))
    ospec = pl.BlockSpec((1, bn, HALF), lambda c, i: (c, i, 0))
    oshape = jax.ShapeDtypeStruct((2, n, HALF), F32)
    return pl.pallas_call(
        _proj3_body,
        grid=(2, n // bn),
        in_specs=[pl.BlockSpec((bn, D), lambda c, i: (i, 0)), wspec, wspec, wspec],
        out_specs=[ospec, ospec, ospec],
        out_shape=[oshape, oshape, oshape],
    )(x, wq, wk, wv)


def _proj1_body(x_ref, w_ref, o_ref):
    o_ref[0] = jnp.dot(x_ref[...], w_ref[...], preferred_element_type=F32)


def _proj1(x, w, bn):
    n = x.shape[0]
    return pl.pallas_call(
        _proj1_body,
        grid=(2, n // bn),
        in_specs=[pl.BlockSpec((bn, D), lambda c, i: (i, 0)),
                  pl.BlockSpec((D, HALF), lambda c, i: (0, c))],
        out_specs=pl.BlockSpec((1, bn, HALF), lambda c, i: (c, i, 0)),
        out_shape=jax.ShapeDtypeStruct((2, n, HALF), F32),
    )(x, w)


def _sc_attention_body(srch, dsth, qt, kt, vt, pet, eattn_o, hattn_o,
                       src_i, dst_i, dst_g, kr, qr, vr, pr, zr,
                       wv_acc, z_acc, sem0, sem1, sem2):
    c = lax.axis_index("c")
    s = lax.axis_index("s")
    cN = c * N
    cE = c * E
    zv = jnp.zeros((16,), F32)
    lane = lax.iota(jnp.int32, 16)

    def _zk(i, _):
        for k in range(HALF // 16):
            kr[i, pl.ds(k * 16, 16)] = zv
        return 0
    lax.fori_loop(0, CH, _zk, 0)

    def _zz(i, _):
        j = i * 16 + lane
        plsc.store_scatter(zr, [j >> 3, j & 7], zv)
        return 0
    lax.fori_loop(0, CH * 8 // 16, _zz, 0)

    for j in range(NPT // CH):
        pltpu.sync_copy(kr, wv_acc.at[pl.ds(s * NPT + j * CH, CH)])
        pltpu.sync_copy(zr, z_acc.at[pl.ds(s * NPT + j * CH, CH)])
    plsc.subcore_barrier()

    def _chunk(ci, _):
        base = s * EPT + ci * CH
        pltpu.sync_copy(srch.at[pl.ds(base, CH)], src_i)
        pltpu.sync_copy(dsth.at[pl.ds(base, CH)], dst_i)

        def _offs(g, _):
            src_i[pl.ds(g * 16, 16)] = src_i[pl.ds(g * 16, 16)] + cN
            dst_g[pl.ds(g * 16, 16)] = dst_i[pl.ds(g * 16, 16)] + cN
            return 0
        lax.fori_loop(0, CH // 16, _offs, 0)

        d0 = pltpu.async_copy(kt.at[src_i], kr, sem0)
        d1 = pltpu.async_copy(qt.at[dst_g], qr, sem1)
        d2 = pltpu.async_copy(vt.at[src_i], vr, sem2)
        pltpu.sync_copy(pet.at[pl.ds(cE + base, CH)], pr)
        d0.wait()
        d1.wait()
        d2.wait()

        for g in range(CH // 16):
            rows = g * 16 + lax.iota(jnp.int32, 16)
            for hh in range(4):
                def _dims(d2i, acc, _hh=hh, _rows=rows):
                    col = jnp.full((16,), _hh * DH, jnp.int32) + d2i
                    kv = plsc.load_gather(kr, [_rows, col])
                    qv = plsc.load_gather(qr, [_rows, col])
                    pv = plsc.load_gather(pr, [_rows, col])
                    sv = kv * qv * pv
                    plsc.store_scatter(kr, [_rows, col], sv)
                    return acc + sv
                acc = lax.fori_loop(0, DH, _dims, zv)
                sexp = jnp.exp(jnp.clip(acc, -5.0, 5.0))
                plsc.store_scatter(zr, [rows, jnp.full((16,), hh, jnp.int32)], sexp)

                def _dims2(d2i, carry, _hh=hh, _rows=rows, _sexp=sexp):
                    col = jnp.full((16,), _hh * DH, jnp.int32) + d2i
                    vv = plsc.load_gather(vr, [_rows, col])
                    plsc.store_scatter(pr, [_rows, col], vv * _sexp)
                    return carry
                lax.fori_loop(0, DH, _dims2, 0)

        pltpu.sync_copy(kr, eattn_o.at[pl.ds(cE + base, CH)])
        pltpu.sync_copy(pr, wv_acc.at[dst_i], add=True)
        pltpu.sync_copy(zr, z_acc.at[dst_i], add=True)
        return 0
    lax.fori_loop(0, NCHUNK, _chunk, 0)
    plsc.subcore_barrier()

    eps = jnp.float32(1e-6)
    for j in range(NPT // CH):
        r0 = s * NPT + j * CH
        pltpu.sync_copy(wv_acc.at[pl.ds(r0, CH)], kr)
        pltpu.sync_copy(z_acc.at[pl.ds(r0, CH)], zr)

        def _div(r, _):
            for head in range(4):
                zb = plsc.load_gather(
                    zr, [jnp.full((16,), r, jnp.int32),
                         jnp.full((16,), head, jnp.int32)])
                rec = jnp.float32(1.0) / (zb + eps)
                for k in (2 * head, 2 * head + 1):
                    kr[r, pl.ds(k * 16, 16)] = kr[r, pl.ds(k * 16, 16)] * rec
            return 0
        lax.fori_loop(0, CH, _div, 0)
        pltpu.sync_copy(kr, hattn_o.at[pl.ds(c * NPAD + r0, CH)])


def _sc_attention(src, dst, qt, kt, vt, pet):
    mesh = plsc.VectorSubcoreMesh(core_axis_name="c", subcore_axis_name="s")
    kern = pl.kernel(
        _sc_attention_body,
        out_type=[
            jax.ShapeDtypeStruct((2 * E, HALF), F32),
            jax.ShapeDtypeStruct((2 * NPAD, HALF), F32),
        ],
        mesh=mesh,
        scratch_types=[
            pltpu.VMEM((CH,), jnp.int32),
            pltpu.VMEM((CH,), jnp.int32),
            pltpu.VMEM((CH,), jnp.int32),
            pltpu.VMEM((CH, HALF), F32),
            pltpu.VMEM((CH, HALF), F32),
            pltpu.VMEM((CH, HALF), F32),
            pltpu.VMEM((CH, HALF), F32),
            pltpu.VMEM((CH, 8), F32),
            pltpu.VMEM_SHARED((NPAD, HALF), F32),
            pltpu.VMEM_SHARED((NPAD, 8), F32),
            pltpu.SemaphoreType.DMA,
            pltpu.SemaphoreType.DMA,
            pltpu.SemaphoreType.DMA,
        ],
        compiler_params=pltpu.CompilerParams(
            needs_layout_passes=False, use_tc_tiling_on_sc=False),
    )
    return kern(src, dst, qt, kt, vt, pet)


def _postA_body(x_ref, aA_ref, aB_ref, wA_ref, wB_ref, b_ref,
                out_ref, s_ref, q_ref):
    i = pl.program_id(0)
    acc = (x_ref[...]
           + jnp.dot(aA_ref[0], wA_ref[...], preferred_element_type=F32)
           + jnp.dot(aB_ref[0], wB_ref[...], preferred_element_type=F32)
           + b_ref[...])
    out_ref[...] = acc

    @pl.when(i == 0)
    def _():
        s_ref[...] = jnp.zeros_like(s_ref)
        q_ref[...] = jnp.zeros_like(q_ref)

    s_ref[...] += jnp.sum(acc, axis=0, keepdims=True)
    q_ref[...] += jnp.sum(acc * acc, axis=0, keepdims=True)


def _postA(x, attn2, wA, wB, b, bn):
    n = x.shape[0]
    sspec = pl.BlockSpec((1, D), lambda i: (0, 0))
    return pl.pallas_call(
        _postA_body,
        grid=(n // bn,),
        in_specs=[pl.BlockSpec((bn, D), lambda i: (i, 0)),
                  pl.BlockSpec((1, bn, HALF), lambda i: (0, i, 0)),
                  pl.BlockSpec((1, bn, HALF), lambda i: (1, i, 0)),
                  pl.BlockSpec((HALF, D), lambda i: (0, 0)),
                  pl.BlockSpec((HALF, D), lambda i: (0, 0)),
                  pl.BlockSpec((1, D), lambda i: (0, 0))],
        out_specs=[pl.BlockSpec((bn, D), lambda i: (i, 0)), sspec, sspec],
        out_shape=[jax.ShapeDtypeStruct((n, D), F32),
                   jax.ShapeDtypeStruct((1, D), F32),
                   jax.ShapeDtypeStruct((1, D), F32)],
    )(x, attn2, attn2, wA, wB, b)


def _postB_body(x_ref, a1_ref, c1_ref, w1_ref, b1_ref, w2_ref, b2_ref,
                out_ref, s_ref, q_ref):
    i = pl.program_id(0)
    x = x_ref[...] * a1_ref[...] + c1_ref[...]
    t = jnp.maximum(jnp.dot(x, w1_ref[...], preferred_element_type=F32)
                    + b1_ref[...], 0.0)
    y = x + jnp.dot(t, w2_ref[...], preferred_element_type=F32) + b2_ref[...]
    out_ref[...] = y

    @pl.when(i == 0)
    def _():
        s_ref[...] = jnp.zeros_like(s_ref)
        q_ref[...] = jnp.zeros_like(q_ref)

    s_ref[...] += jnp.sum(y, axis=0, keepdims=True)
    q_ref[...] += jnp.sum(y * y, axis=0, keepdims=True)


def _postB(x, a1, c1, w1, b1, w2, b2, bn):
    n = x.shape[0]
    sspec = pl.BlockSpec((1, D), lambda i: (0, 0))
    return pl.pallas_call(
        _postB_body,
        grid=(n // bn,),
        in_specs=[pl.BlockSpec((bn, D), lambda i: (i, 0)),
                  pl.BlockSpec((1, D), lambda i: (0, 0)),
                  pl.BlockSpec((1, D), lambda i: (0, 0)),
                  pl.BlockSpec((D, 2 * D), lambda i: (0, 0)),
                  pl.BlockSpec((1, 2 * D), lambda i: (0, 0)),
                  pl.BlockSpec((2 * D, D), lambda i: (0, 0)),
                  pl.BlockSpec((1, D), lambda i: (0, 0))],
        out_specs=[pl.BlockSpec((bn, D), lambda i: (i, 0)), sspec, sspec],
        out_shape=[jax.ShapeDtypeStruct((n, D), F32),
                   jax.ShapeDtypeStruct((1, D), F32),
                   jax.ShapeDtypeStruct((1, D), F32)],
    )(x, a1, c1, w1, b1, w2, b2)


def _postC_body(x_ref, a_ref, c_ref, out_ref):
    out_ref[...] = x_ref[...] * a_ref[...] + c_ref[...]


def _postC(x, a, c, bn):
    n = x.shape[0]
    return pl.pallas_call(
        _postC_body,
        grid=(n // bn,),
        in_specs=[pl.BlockSpec((bn, D), lambda i: (i, 0)),
                  pl.BlockSpec((1, D), lambda i: (0, 0)),
                  pl.BlockSpec((1, D), lambda i: (0, 0))],
        out_specs=pl.BlockSpec((bn, D), lambda i: (i, 0)),
        out_shape=jax.ShapeDtypeStruct((n, D), F32),
    )(x, a, c)


def _bn_coef(ssum, sqsum, n, g, b):
    mu = ssum[0] / n
    var = sqsum[0] / n - mu * mu
    a = g / jnp.sqrt(var + 1e-5)
    return a.reshape(1, D), (b - mu * a).reshape(1, D)


def kernel(h, e, edge_index, WQ, WK, WV, We, WOh, bOh, WOe, bOe,
           W1h, b1h, W2h, b2h, W1e, b1e, W2e, b2e,
           g1h, be1h, g1e, be1e, g2h, be2h, g2e, be2e):
    q2, k2, v2 = _proj3(h, WQ, WK, WV, bn=2000)
    pe2 = _proj1(e, We, bn=2000)

    eattn, hattn = _sc_attention(
        edge_index[0], edge_index[1],
        q2.reshape(2 * N, HALF), k2.reshape(2 * N, HALF),
        v2.reshape(2 * N, HALF), pe2.reshape(2 * E, HALF))

    hattn2 = hattn.reshape(2, NPAD, HALF)
    eattn2 = eattn.reshape(2, E, HALF)

    hh, hs1, hq1 = _postA(h, hattn2, WOh[:HALF], WOh[HALF:],
                          bOh.reshape(1, D), bn=2000)
    ee, es1, eq1 = _postA(e, eattn2, WOe[:HALF], WOe[HALF:],
                          bOe.reshape(1, D), bn=2000)

    ha1, hc1 = _bn_coef(hs1, hq1, N, g1h, be1h)
    ea1, ec1 = _bn_coef(es1, eq1, E, g1e, be1e)

    hy, hs2, hq2 = _postB(hh, ha1, hc1, W1h, b1h.reshape(1, 2 * D),
                          W2h, b2h.reshape(1, D), bn=2000)
    ey, es2, eq2 = _postB(ee, ea1, ec1, W1e, b1e.reshape(1, 2 * D),
                          W2e, b2e.reshape(1, D), bn=2000)

    ha2, hc2 = _bn_coef(hs2, hq2, N, g2h, be2h)
    ea2, ec2 = _bn_coef(es2, eq2, E, g2e, be2e)

    h_out = _postC(hy, ha2, hc2, bn=2000)
    e_out = _postC(ey, ea2, ec2, bn=2000)
    return (h_out, e_out)


# speedup vs baseline: 5.4479x; 1.4823x over previous
import functools

import jax
import jax.numpy as jnp
from jax import lax
from jax.experimental import pallas as pl
from jax.experimental.pallas import tpu as pltpu
from jax.experimental.pallas import tpu_sc as plsc

N = 10000
E = 160000
D = 256
H = 8
DH = 32
HALF = 128
NS = 16
EPT = E // NS
CH = 80
NCHUNK = EPT // CH
NPAD = 10240
NPT = NPAD // NS
F32 = jnp.float32


def _proj3_body(x_ref, wq_ref, wk_ref, wv_ref, q_ref, k_ref, v_ref):
    x = x_ref[...]
    scale = jnp.float32(1.0 / (DH ** 0.5))
    q_ref[0] = jnp.dot(x, wq_ref[...], preferred_element_type=F32) * scale
    k_ref[0] = jnp.dot(x, wk_ref[...], preferred_element_type=F32)
    v_ref[0] = jnp.dot(x, wv_ref[...], preferred_element_type=F32)


def _proj3(x, wq, wk, wv, bn):
    n = x.shape[0]
    wspec = pl.BlockSpec((D, HALF), lambda c, i: (0, c))
    ospec = pl.BlockSpec((1, bn, HALF), lambda c, i: (c, i, 0))
    oshape = jax.ShapeDtypeStruct((2, n, HALF), F32)
    return pl.pallas_call(
        _proj3_body,
        grid=(2, n // bn),
        in_specs=[pl.BlockSpec((bn, D), lambda c, i: (i, 0)), wspec, wspec, wspec],
        out_specs=[ospec, ospec, ospec],
        out_shape=[oshape, oshape, oshape],
    )(x, wq, wk, wv)


def _proj1_body(x_ref, w_ref, o_ref):
    o_ref[0] = jnp.dot(x_ref[...], w_ref[...], preferred_element_type=F32)


def _proj1(x, w, bn):
    n = x.shape[0]
    return pl.pallas_call(
        _proj1_body,
        grid=(2, n // bn),
        in_specs=[pl.BlockSpec((bn, D), lambda c, i: (i, 0)),
                  pl.BlockSpec((D, HALF), lambda c, i: (0, c))],
        out_specs=pl.BlockSpec((1, bn, HALF), lambda c, i: (c, i, 0)),
        out_shape=jax.ShapeDtypeStruct((2, n, HALF), F32),
    )(x, w)


def _sc_attention_body(srch, dsth, qt, kt, vt, pet, eattn_o, hattn_o,
                       src_i, dst_i, dst_g, kr, qr, vr, pr, zr,
                       wv_acc, z_acc, sem0, sem1, sem2):
    c = lax.axis_index("c")
    s = lax.axis_index("s")
    cN = c * N
    cE = c * E
    zv = jnp.zeros((16,), F32)
    lane = lax.iota(jnp.int32, 16)

    def _zk(i, _):
        for k in range(HALF // 16):
            kr[i, pl.ds(k * 16, 16)] = zv
        return 0
    lax.fori_loop(0, CH, _zk, 0)

    def _zz(i, _):
        j = i * 16 + lane
        plsc.store_scatter(zr, [j >> 3, j & 7], zv)
        return 0
    lax.fori_loop(0, CH * 8 // 16, _zz, 0)

    for j in range(NPT // CH):
        pltpu.sync_copy(kr, wv_acc.at[pl.ds(s * NPT + j * CH, CH)])
        pltpu.sync_copy(zr, z_acc.at[pl.ds(s * NPT + j * CH, CH)])
    plsc.subcore_barrier()

    def _chunk(ci, _):
        base = s * EPT + ci * CH
        pltpu.sync_copy(srch.at[pl.ds(base, CH)], src_i)
        pltpu.sync_copy(dsth.at[pl.ds(base, CH)], dst_i)

        def _offs(g, _):
            src_i[pl.ds(g * 16, 16)] = src_i[pl.ds(g * 16, 16)] + cN
            dst_g[pl.ds(g * 16, 16)] = dst_i[pl.ds(g * 16, 16)] + cN
            return 0
        lax.fori_loop(0, CH // 16, _offs, 0)

        d0 = pltpu.async_copy(kt.at[src_i], kr, sem0)
        d1 = pltpu.async_copy(qt.at[dst_g], qr, sem1)
        d2 = pltpu.async_copy(vt.at[src_i], vr, sem2)
        pltpu.sync_copy(pet.at[pl.ds(cE + base, CH)], pr)
        d0.wait()
        d1.wait()
        d2.wait()

        for g in range(CH // 16):
            rows = g * 16 + lax.iota(jnp.int32, 16)
            for hh in range(4):
                @plsc.parallel_loop(0, DH, carry=zv, unroll=8)
                def acc(d2i, acc, _hh=hh, _rows=rows):
                    col = jnp.full((16,), _hh * DH, jnp.int32) + d2i
                    kv = plsc.load_gather(kr, [_rows, col])
                    qv = plsc.load_gather(qr, [_rows, col])
                    pv = plsc.load_gather(pr, [_rows, col])
                    sv = kv * qv * pv
                    plsc.store_scatter(kr, [_rows, col], sv)
                    return acc + sv
                sexp = jnp.exp(jnp.clip(acc, -5.0, 5.0))
                plsc.store_scatter(zr, [rows, jnp.full((16,), hh, jnp.int32)], sexp)

                @plsc.parallel_loop(0, DH, unroll=8)
                def _dims2(d2i, _hh=hh, _rows=rows, _sexp=sexp):
                    col = jnp.full((16,), _hh * DH, jnp.int32) + d2i
                    vv = plsc.load_gather(vr, [_rows, col])
                    plsc.store_scatter(pr, [_rows, col], vv * _sexp)

        pltpu.sync_copy(kr, eattn_o.at[pl.ds(cE + base, CH)])
        pltpu.sync_copy(pr, wv_acc.at[dst_i], add=True)
        pltpu.sync_copy(zr, z_acc.at[dst_i], add=True)
        return 0
    lax.fori_loop(0, NCHUNK, _chunk, 0)
    plsc.subcore_barrier()

    eps = jnp.float32(1e-6)
    for j in range(NPT // CH):
        r0 = s * NPT + j * CH
        pltpu.sync_copy(wv_acc.at[pl.ds(r0, CH)], kr)
        pltpu.sync_copy(z_acc.at[pl.ds(r0, CH)], zr)

        @plsc.parallel_loop(0, CH, unroll=4)
        def _div(r):
            for head in range(4):
                zb = plsc.load_gather(
                    zr, [jnp.full((16,), r, jnp.int32),
                         jnp.full((16,), head, jnp.int32)])
                rec = jnp.float32(1.0) / (zb + eps)
                for k in (2 * head, 2 * head + 1):
                    kr[r, pl.ds(k * 16, 16)] = kr[r, pl.ds(k * 16, 16)] * rec
        pltpu.sync_copy(kr, hattn_o.at[pl.ds(c * NPAD + r0, CH)])


def _sc_attention(src, dst, qt, kt, vt, pet):
    mesh = plsc.VectorSubcoreMesh(core_axis_name="c", subcore_axis_name="s")
    kern = pl.kernel(
        _sc_attention_body,
        out_type=[
            jax.ShapeDtypeStruct((2 * E, HALF), F32),
            jax.ShapeDtypeStruct((2 * NPAD, HALF), F32),
        ],
        mesh=mesh,
        scratch_types=[
            pltpu.VMEM((CH,), jnp.int32),
            pltpu.VMEM((CH,), jnp.int32),
            pltpu.VMEM((CH,), jnp.int32),
            pltpu.VMEM((CH, HALF), F32),
            pltpu.VMEM((CH, HALF), F32),
            pltpu.VMEM((CH, HALF), F32),
            pltpu.VMEM((CH, HALF), F32),
            pltpu.VMEM((CH, 8), F32),
            pltpu.VMEM_SHARED((NPAD, HALF), F32),
            pltpu.VMEM_SHARED((NPAD, 8), F32),
            pltpu.SemaphoreType.DMA,
            pltpu.SemaphoreType.DMA,
            pltpu.SemaphoreType.DMA,
        ],
        compiler_params=pltpu.CompilerParams(
            needs_layout_passes=False, use_tc_tiling_on_sc=False),
    )
    return kern(src, dst, qt, kt, vt, pet)


def _postA_body(x_ref, aA_ref, aB_ref, wA_ref, wB_ref, b_ref,
                out_ref, s_ref, q_ref):
    i = pl.program_id(0)
    acc = (x_ref[...]
           + jnp.dot(aA_ref[0], wA_ref[...], preferred_element_type=F32)
           + jnp.dot(aB_ref[0], wB_ref[...], preferred_element_type=F32)
           + b_ref[...])
    out_ref[...] = acc

    @pl.when(i == 0)
    def _():
        s_ref[...] = jnp.zeros_like(s_ref)
        q_ref[...] = jnp.zeros_like(q_ref)

    s_ref[...] += jnp.sum(acc, axis=0, keepdims=True)
    q_ref[...] += jnp.sum(acc * acc, axis=0, keepdims=True)


def _postA(x, attn2, wA, wB, b, bn):
    n = x.shape[0]
    sspec = pl.BlockSpec((1, D), lambda i: (0, 0))
    return pl.pallas_call(
        _postA_body,
        grid=(n // bn,),
        in_specs=[pl.BlockSpec((bn, D), lambda i: (i, 0)),
                  pl.BlockSpec((1, bn, HALF), lambda i: (0, i, 0)),
                  pl.BlockSpec((1, bn, HALF), lambda i: (1, i, 0)),
                  pl.BlockSpec((HALF, D), lambda i: (0, 0)),
                  pl.BlockSpec((HALF, D), lambda i: (0, 0)),
                  pl.BlockSpec((1, D), lambda i: (0, 0))],
        out_specs=[pl.BlockSpec((bn, D), lambda i: (i, 0)), sspec, sspec],
        out_shape=[jax.ShapeDtypeStruct((n, D), F32),
                   jax.ShapeDtypeStruct((1, D), F32),
                   jax.ShapeDtypeStruct((1, D), F32)],
    )(x, attn2, attn2, wA, wB, b)


def _postB_body(x_ref, a1_ref, c1_ref, w1_ref, b1_ref, w2_ref, b2_ref,
                out_ref, s_ref, q_ref):
    i = pl.program_id(0)
    x = x_ref[...] * a1_ref[...] + c1_ref[...]
    t = jnp.maximum(jnp.dot(x, w1_ref[...], preferred_element_type=F32)
                    + b1_ref[...], 0.0)
    y = x + jnp.dot(t, w2_ref[...], preferred_element_type=F32) + b2_ref[...]
    out_ref[...] = y

    @pl.when(i == 0)
    def _():
        s_ref[...] = jnp.zeros_like(s_ref)
        q_ref[...] = jnp.zeros_like(q_ref)

    s_ref[...] += jnp.sum(y, axis=0, keepdims=True)
    q_ref[...] += jnp.sum(y * y, axis=0, keepdims=True)


def _postB(x, a1, c1, w1, b1, w2, b2, bn):
    n = x.shape[0]
    sspec = pl.BlockSpec((1, D), lambda i: (0, 0))
    return pl.pallas_call(
        _postB_body,
        grid=(n // bn,),
        in_specs=[pl.BlockSpec((bn, D), lambda i: (i, 0)),
                  pl.BlockSpec((1, D), lambda i: (0, 0)),
                  pl.BlockSpec((1, D), lambda i: (0, 0)),
                  pl.BlockSpec((D, 2 * D), lambda i: (0, 0)),
                  pl.BlockSpec((1, 2 * D), lambda i: (0, 0)),
                  pl.BlockSpec((2 * D, D), lambda i: (0, 0)),
                  pl.BlockSpec((1, D), lambda i: (0, 0))],
        out_specs=[pl.BlockSpec((bn, D), lambda i: (i, 0)), sspec, sspec],
        out_shape=[jax.ShapeDtypeStruct((n, D), F32),
                   jax.ShapeDtypeStruct((1, D), F32),
                   jax.ShapeDtypeStruct((1, D), F32)],
    )(x, a1, c1, w1, b1, w2, b2)


def _postC_body(x_ref, a_ref, c_ref, out_ref):
    out_ref[...] = x_ref[...] * a_ref[...] + c_ref[...]


def _postC(x, a, c, bn):
    n = x.shape[0]
    return pl.pallas_call(
        _postC_body,
        grid=(n // bn,),
        in_specs=[pl.BlockSpec((bn, D), lambda i: (i, 0)),
                  pl.BlockSpec((1, D), lambda i: (0, 0)),
                  pl.BlockSpec((1, D), lambda i: (0, 0))],
        out_specs=pl.BlockSpec((bn, D), lambda i: (i, 0)),
        out_shape=jax.ShapeDtypeStruct((n, D), F32),
    )(x, a, c)


def _bn_coef(ssum, sqsum, n, g, b):
    mu = ssum[0] / n
    var = sqsum[0] / n - mu * mu
    a = g / jnp.sqrt(var + 1e-5)
    return a.reshape(1, D), (b - mu * a).reshape(1, D)


def kernel(h, e, edge_index, WQ, WK, WV, We, WOh, bOh, WOe, bOe,
           W1h, b1h, W2h, b2h, W1e, b1e, W2e, b2e,
           g1h, be1h, g1e, be1e, g2h, be2h, g2e, be2e):
    q2, k2, v2 = _proj3(h, WQ, WK, WV, bn=2000)
    pe2 = _proj1(e, We, bn=2000)

    eattn, hattn = _sc_attention(
        edge_index[0], edge_index[1],
        q2.reshape(2 * N, HALF), k2.reshape(2 * N, HALF),
        v2.reshape(2 * N, HALF), pe2.reshape(2 * E, HALF))

    hattn2 = hattn.reshape(2, NPAD, HALF)
    eattn2 = eattn.reshape(2, E, HALF)

    hh, hs1, hq1 = _postA(h, hattn2, WOh[:HALF], WOh[HALF:],
                          bOh.reshape(1, D), bn=2000)
    ee, es1, eq1 = _postA(e, eattn2, WOe[:HALF], WOe[HALF:],
                          bOe.reshape(1, D), bn=2000)

    ha1, hc1 = _bn_coef(hs1, hq1, N, g1h, be1h)
    ea1, ec1 = _bn_coef(es1, eq1, E, g1e, be1e)

    hy, hs2, hq2 = _postB(hh, ha1, hc1, W1h, b1h.reshape(1, 2 * D),
                          W2h, b2h.reshape(1, D), bn=2000)
    ey, es2, eq2 = _postB(ee, ea1, ec1, W1e, b1e.reshape(1, 2 * D),
                          W2e, b2e.reshape(1, D), bn=2000)

    ha2, hc2 = _bn_coef(hs2, hq2, N, g2h, be2h)
    ea2, ec2 = _bn_coef(es2, eq2, E, g2e, be2e)

    h_out = _postC(hy, ha2, hc2, bn=2000)
    e_out = _postC(ey, ea2, ec2, bn=2000)
    return (h_out, e_out)


# speedup vs baseline: 5.6445x; 1.0361x over previous
import functools

import jax
import jax.numpy as jnp
from jax import lax
from jax.experimental import pallas as pl
from jax.experimental.pallas import tpu as pltpu
from jax.experimental.pallas import tpu_sc as plsc

N = 10000
E = 160000
D = 256
H = 8
DH = 32
HALF = 128
NS = 16
EPT = E // NS
CH = 16
NPAD = 10240
NPT = NPAD // NS
F32 = jnp.float32


def _proj3_body(x_ref, wq_ref, wk_ref, wv_ref, q_ref, kv_ref):
    x = x_ref[...]
    scale = jnp.float32(1.0 / (DH ** 0.5))
    q_ref[0] = jnp.dot(x, wq_ref[...], preferred_element_type=F32) * scale
    kv_ref[0] = jnp.concatenate(
        [jnp.dot(x, wk_ref[...], preferred_element_type=F32),
         jnp.dot(x, wv_ref[...], preferred_element_type=F32)], axis=1)


def _proj3(x, wq, wk, wv, bn):
    n = x.shape[0]
    wspec = pl.BlockSpec((D, HALF), lambda c, i: (0, c))
    return pl.pallas_call(
        _proj3_body,
        grid=(2, n // bn),
        in_specs=[pl.BlockSpec((bn, D), lambda c, i: (i, 0)), wspec, wspec, wspec],
        out_specs=[pl.BlockSpec((1, bn, HALF), lambda c, i: (c, i, 0)),
                   pl.BlockSpec((1, bn, 2 * HALF), lambda c, i: (c, i, 0))],
        out_shape=[jax.ShapeDtypeStruct((2, n, HALF), F32),
                   jax.ShapeDtypeStruct((2, n, 2 * HALF), F32)],
    )(x, wq, wk, wv)


def _proj1_body(x_ref, w_ref, o_ref):
    o_ref[0] = jnp.dot(x_ref[...], w_ref[...], preferred_element_type=F32)


def _proj1(x, w, bn):
    n = x.shape[0]
    return pl.pallas_call(
        _proj1_body,
        grid=(2, n // bn),
        in_specs=[pl.BlockSpec((bn, D), lambda c, i: (i, 0)),
                  pl.BlockSpec((D, HALF), lambda c, i: (0, c))],
        out_specs=pl.BlockSpec((1, bn, HALF), lambda c, i: (c, i, 0)),
        out_shape=jax.ShapeDtypeStruct((2, n, HALF), F32),
    )(x, w)


G = 5


def _sc_attention_body(srch, dsth, kvt, qt, pet, eattn_o, hattn_o,
                       *refs):
    src_i = refs[0:G]
    dst_i = refs[G:2 * G]
    dst_g = refs[2 * G:3 * G]
    kvr = refs[3 * G:4 * G]
    qr = refs[4 * G:5 * G]
    pr = refs[5 * G:6 * G]
    zr = refs[6 * G:7 * G]
    sem_si = refs[7 * G:8 * G]
    sem_di = refs[8 * G:9 * G]
    sem_kv = refs[9 * G:10 * G]
    sem_q = refs[10 * G:11 * G]
    sem_pe = refs[11 * G:12 * G]
    sem_eo = refs[12 * G:13 * G]
    sem_wo = refs[13 * G:14 * G]
    sem_zo = refs[14 * G:15 * G]
    wv_acc = refs[15 * G]
    z_acc = refs[15 * G + 1]

    c = lax.axis_index("c")
    s = lax.axis_index("s")
    cN = c * N
    cE = c * E
    zv = jnp.zeros((16,), F32)
    lane = lax.iota(jnp.int32, 16)
    NT = (E // CH) // NS

    for b in range(G):
        @plsc.parallel_loop(0, CH, unroll=4)
        def _zk(i, _b=b):
            for k in range(HALF // 16):
                qr[_b][i, pl.ds(k * 16, 16)] = zv

        @plsc.parallel_loop(0, CH * 8 // 16, unroll=2)
        def _zz(i, _b=b):
            j = i * 16 + lane
            plsc.store_scatter(zr[_b], [j >> 3, j & 7], zv)

    for j in range(NPT // CH):
        r0 = s * NPT + j * CH
        pltpu.sync_copy(qr[0], wv_acc.at[pl.ds(r0, CH)])
        pltpu.sync_copy(zr[0], z_acc.at[pl.ds(r0, CH)])
    plsc.subcore_barrier()

    def _group(gi, _):
        t0 = gi * G
        idx_d = []
        for b in range(G):
            base = (s + NS * (t0 + b)) * CH
            idx_d.append(
                pltpu.async_copy(srch.at[pl.ds(base, CH)], src_i[b], sem_si[b]))
            idx_d.append(
                pltpu.async_copy(dsth.at[pl.ds(base, CH)], dst_i[b], sem_di[b]))
        for d in idx_d:
            d.wait()
        in_d = []
        for b in range(G):
            base = (s + NS * (t0 + b)) * CH
            src_i[b][pl.ds(0, 16)] = src_i[b][pl.ds(0, 16)] + cN
            dst_g[b][pl.ds(0, 16)] = dst_i[b][pl.ds(0, 16)] + cN
            in_d.append((
                pltpu.async_copy(kvt.at[src_i[b]], kvr[b], sem_kv[b]),
                pltpu.async_copy(qt.at[dst_g[b]], qr[b], sem_q[b]),
                pltpu.async_copy(pet.at[pl.ds(cE + base, CH)], pr[b], sem_pe[b]),
            ))
        out_d = []
        for b in range(G):
            base = (s + NS * (t0 + b)) * CH
            for d in in_d[b]:
                d.wait()
            for hh in range(4):
                @plsc.parallel_loop(0, DH, carry=zv, unroll=8)
                def acc(d2i, a, _hh=hh, _b=b):
                    col = jnp.full((16,), _hh * DH, jnp.int32) + d2i
                    kv = plsc.load_gather(kvr[_b], [lane, col])
                    qv = plsc.load_gather(qr[_b], [lane, col])
                    pv = plsc.load_gather(pr[_b], [lane, col])
                    sv = kv * qv * pv
                    plsc.store_scatter(qr[_b], [lane, col], sv)
                    return a + sv
                sexp = jnp.exp(jnp.clip(acc, -5.0, 5.0))
                plsc.store_scatter(zr[b], [lane, jnp.full((16,), hh, jnp.int32)], sexp)

                @plsc.parallel_loop(0, DH, unroll=8)
                def _dims2(d2i, _hh=hh, _sexp=sexp, _b=b):
                    col = jnp.full((16,), _hh * DH, jnp.int32) + d2i
                    vv = plsc.load_gather(kvr[_b], [lane, col + HALF])
                    plsc.store_scatter(pr[_b], [lane, col], vv * _sexp)

            out_d.append((
                pltpu.async_copy(qr[b], eattn_o.at[pl.ds(cE + base, CH)], sem_eo[b]),
            ))
            pltpu.sync_copy(pr[b], wv_acc.at[dst_i[b]], add=True)
            pltpu.sync_copy(zr[b], z_acc.at[dst_i[b]], add=True)
        for ds3 in out_d:
            for d in ds3:
                d.wait()
        return 0
    lax.fori_loop(0, NT // G, _group, 0)
    plsc.subcore_barrier()

    eps = jnp.float32(1e-6)

    def _divchunk(j, _):
        r0 = s * NPT + j * CH
        pltpu.sync_copy(wv_acc.at[pl.ds(r0, CH)], qr[0])
        pltpu.sync_copy(z_acc.at[pl.ds(r0, CH)], zr[0])

        @plsc.parallel_loop(0, CH, unroll=4)
        def _div(r):
            for head in range(4):
                zb = plsc.load_gather(
                    zr[0], [jnp.full((16,), r, jnp.int32),
                            jnp.full((16,), head, jnp.int32)])
                rec = jnp.float32(1.0) / (zb + eps)
                for k in (2 * head, 2 * head + 1):
                    qr[0][r, pl.ds(k * 16, 16)] = qr[0][r, pl.ds(k * 16, 16)] * rec
        pltpu.sync_copy(qr[0], hattn_o.at[pl.ds(c * NPAD + r0, CH)])
        return 0
    lax.fori_loop(0, NPT // CH, _divchunk, 0)


def _sc_attention(src, dst, kvt, qt, pet):
    mesh = plsc.VectorSubcoreMesh(core_axis_name="c", subcore_axis_name="s")
    scratch = (
        [pltpu.VMEM((CH,), jnp.int32) for _ in range(G)]
        + [pltpu.VMEM((CH,), jnp.int32) for _ in range(G)]
        + [pltpu.VMEM((CH,), jnp.int32) for _ in range(G)]
        + [pltpu.VMEM((CH, 2 * HALF), F32) for _ in range(G)]
        + [pltpu.VMEM((CH, HALF), F32) for _ in range(G)]
        + [pltpu.VMEM((CH, HALF), F32) for _ in range(G)]
        + [pltpu.VMEM((CH, 8), F32) for _ in range(G)]
        + [pltpu.SemaphoreType.DMA for _ in range(8 * G)]
        + [pltpu.VMEM_SHARED((NPAD, HALF), F32),
           pltpu.VMEM_SHARED((NPAD, 8), F32)]
    )
    kern = pl.kernel(
        _sc_attention_body,
        out_type=[
            jax.ShapeDtypeStruct((2 * E, HALF), F32),
            jax.ShapeDtypeStruct((2 * NPAD, HALF), F32),
        ],
        mesh=mesh,
        scratch_types=scratch,
        compiler_params=pltpu.CompilerParams(
            needs_layout_passes=False, use_tc_tiling_on_sc=False),
    )
    return kern(src, dst, kvt, qt, pet)


def _postA_body(x_ref, aA_ref, aB_ref, wA_ref, wB_ref, b_ref,
                out_ref, s_ref, q_ref):
    i = pl.program_id(0)
    acc = (x_ref[...]
           + jnp.dot(aA_ref[0], wA_ref[...], preferred_element_type=F32)
           + jnp.dot(aB_ref[0], wB_ref[...], preferred_element_type=F32)
           + b_ref[...])
    out_ref[...] = acc

    @pl.when(i == 0)
    def _():
        s_ref[...] = jnp.zeros_like(s_ref)
        q_ref[...] = jnp.zeros_like(q_ref)

    s_ref[...] += jnp.sum(acc, axis=0, keepdims=True)
    q_ref[...] += jnp.sum(acc * acc, axis=0, keepdims=True)


def _postA(x, attn2, wA, wB, b, bn):
    n = x.shape[0]
    sspec = pl.BlockSpec((1, D), lambda i: (0, 0))
    return pl.pallas_call(
        _postA_body,
        grid=(n // bn,),
        in_specs=[pl.BlockSpec((bn, D), lambda i: (i, 0)),
                  pl.BlockSpec((1, bn, HALF), lambda i: (0, i, 0)),
                  pl.BlockSpec((1, bn, HALF), lambda i: (1, i, 0)),
                  pl.BlockSpec((HALF, D), lambda i: (0, 0)),
                  pl.BlockSpec((HALF, D), lambda i: (0, 0)),
                  pl.BlockSpec((1, D), lambda i: (0, 0))],
        out_specs=[pl.BlockSpec((bn, D), lambda i: (i, 0)), sspec, sspec],
        out_shape=[jax.ShapeDtypeStruct((n, D), F32),
                   jax.ShapeDtypeStruct((1, D), F32),
                   jax.ShapeDtypeStruct((1, D), F32)],
    )(x, attn2, attn2, wA, wB, b)


def _postB_body(x_ref, a1_ref, c1_ref, w1_ref, b1_ref, w2_ref, b2_ref,
                out_ref, s_ref, q_ref):
    i = pl.program_id(0)
    x = x_ref[...] * a1_ref[...] + c1_ref[...]
    t = jnp.maximum(jnp.dot(x, w1_ref[...], preferred_element_type=F32)
                    + b1_ref[...], 0.0)
    y = x + jnp.dot(t, w2_ref[...], preferred_element_type=F32) + b2_ref[...]
    out_ref[...] = y

    @pl.when(i == 0)
    def _():
        s_ref[...] = jnp.zeros_like(s_ref)
        q_ref[...] = jnp.zeros_like(q_ref)

    s_ref[...] += jnp.sum(y, axis=0, keepdims=True)
    q_ref[...] += jnp.sum(y * y, axis=0, keepdims=True)


def _postB(x, a1, c1, w1, b1, w2, b2, bn):
    n = x.shape[0]
    sspec = pl.BlockSpec((1, D), lambda i: (0, 0))
    return pl.pallas_call(
        _postB_body,
        grid=(n // bn,),
        in_specs=[pl.BlockSpec((bn, D), lambda i: (i, 0)),
                  pl.BlockSpec((1, D), lambda i: (0, 0)),
                  pl.BlockSpec((1, D), lambda i: (0, 0)),
                  pl.BlockSpec((D, 2 * D), lambda i: (0, 0)),
                  pl.BlockSpec((1, 2 * D), lambda i: (0, 0)),
                  pl.BlockSpec((2 * D, D), lambda i: (0, 0)),
                  pl.BlockSpec((1, D), lambda i: (0, 0))],
        out_specs=[pl.BlockSpec((bn, D), lambda i: (i, 0)), sspec, sspec],
        out_shape=[jax.ShapeDtypeStruct((n, D), F32),
                   jax.ShapeDtypeStruct((1, D), F32),
                   jax.ShapeDtypeStruct((1, D), F32)],
    )(x, a1, c1, w1, b1, w2, b2)


def _postC_body(x_ref, a_ref, c_ref, out_ref):
    out_ref[...] = x_ref[...] * a_ref[...] + c_ref[...]


def _postC(x, a, c, bn):
    n = x.shape[0]
    return pl.pallas_call(
        _postC_body,
        grid=(n // bn,),
        in_specs=[pl.BlockSpec((bn, D), lambda i: (i, 0)),
                  pl.BlockSpec((1, D), lambda i: (0, 0)),
                  pl.BlockSpec((1, D), lambda i: (0, 0))],
        out_specs=pl.BlockSpec((bn, D), lambda i: (i, 0)),
        out_shape=jax.ShapeDtypeStruct((n, D), F32),
    )(x, a, c)


def _bn_coef(ssum, sqsum, n, g, b):
    mu = ssum[0] / n
    var = sqsum[0] / n - mu * mu
    a = g / jnp.sqrt(var + 1e-5)
    return a.reshape(1, D), (b - mu * a).reshape(1, D)


def kernel(h, e, edge_index, WQ, WK, WV, We, WOh, bOh, WOe, bOe,
           W1h, b1h, W2h, b2h, W1e, b1e, W2e, b2e,
           g1h, be1h, g1e, be1e, g2h, be2h, g2e, be2e):
    q2, kv2 = _proj3(h, WQ, WK, WV, bn=2000)
    pe2 = _proj1(e, We, bn=2000)

    eattn, hattn = _sc_attention(
        edge_index[0], edge_index[1],
        kv2.reshape(2 * N, 2 * HALF), q2.reshape(2 * N, HALF),
        pe2.reshape(2 * E, HALF))

    hattn2 = hattn.reshape(2, NPAD, HALF)
    eattn2 = eattn.reshape(2, E, HALF)

    hh, hs1, hq1 = _postA(h, hattn2, WOh[:HALF], WOh[HALF:],
                          bOh.reshape(1, D), bn=2000)
    ee, es1, eq1 = _postA(e, eattn2, WOe[:HALF], WOe[HALF:],
                          bOe.reshape(1, D), bn=2000)

    ha1, hc1 = _bn_coef(hs1, hq1, N, g1h, be1h)
    ea1, ec1 = _bn_coef(es1, eq1, E, g1e, be1e)

    hy, hs2, hq2 = _postB(hh, ha1, hc1, W1h, b1h.reshape(1, 2 * D),
                          W2h, b2h.reshape(1, D), bn=2000)
    ey, es2, eq2 = _postB(ee, ea1, ec1, W1e, b1e.reshape(1, 2 * D),
                          W2e, b2e.reshape(1, D), bn=2000)

    ha2, hc2 = _bn_coef(hs2, hq2, N, g2h, be2h)
    ea2, ec2 = _bn_coef(es2, eq2, E, g2e, be2e)

    h_out = _postC(hy, ha2, hc2, bn=2000)
    e_out = _postC(ey, ea2, ec2, bn=2000)
    return (h_out, e_out)


# speedup vs baseline: 5.7233x; 1.0140x over previous
import functools

import jax
import jax.numpy as jnp
from jax import lax
from jax.experimental import pallas as pl
from jax.experimental.pallas import tpu as pltpu
from jax.experimental.pallas import tpu_sc as plsc

N = 10000
E = 160000
D = 256
H = 8
DH = 32
HALF = 128
NS = 16
EPT = E // NS
CH = 16
NPAD = 10240
NPT = NPAD // NS
F32 = jnp.float32


def _proj3_body(x_ref, wq_ref, wk_ref, wv_ref, q_ref, kv_ref):
    x = x_ref[...]
    scale = jnp.float32(1.0 / (DH ** 0.5))
    q_ref[0] = jnp.dot(x, wq_ref[...], preferred_element_type=F32) * scale
    kv_ref[0] = jnp.concatenate(
        [jnp.dot(x, wk_ref[...], preferred_element_type=F32),
         jnp.dot(x, wv_ref[...], preferred_element_type=F32)], axis=1)


def _proj3(x, wq, wk, wv, bn):
    n = x.shape[0]
    wspec = pl.BlockSpec((D, HALF), lambda c, i: (0, c))
    return pl.pallas_call(
        _proj3_body,
        grid=(2, n // bn),
        in_specs=[pl.BlockSpec((bn, D), lambda c, i: (i, 0)), wspec, wspec, wspec],
        out_specs=[pl.BlockSpec((1, bn, HALF), lambda c, i: (c, i, 0)),
                   pl.BlockSpec((1, bn, 2 * HALF), lambda c, i: (c, i, 0))],
        out_shape=[jax.ShapeDtypeStruct((2, n, HALF), F32),
                   jax.ShapeDtypeStruct((2, n, 2 * HALF), F32)],
    )(x, wq, wk, wv)


def _proj1_body(x_ref, w_ref, o_ref):
    o_ref[0] = jnp.dot(x_ref[...], w_ref[...], preferred_element_type=F32)


def _proj1(x, w, bn):
    n = x.shape[0]
    return pl.pallas_call(
        _proj1_body,
        grid=(2, n // bn),
        in_specs=[pl.BlockSpec((bn, D), lambda c, i: (i, 0)),
                  pl.BlockSpec((D, HALF), lambda c, i: (0, c))],
        out_specs=pl.BlockSpec((1, bn, HALF), lambda c, i: (c, i, 0)),
        out_shape=jax.ShapeDtypeStruct((2, n, HALF), F32),
    )(x, w)


G = 4
GE = G * CH


def _sc_attention_body(srch, dsth, kvt, qt, pet, eattn_o, hattn_o,
                       *refs):
    src_i = refs[0:G]
    dst_i = refs[G:2 * G]
    dst_g = refs[2 * G:3 * G]
    kvr = refs[3 * G:4 * G]
    qr = refs[4 * G:5 * G]
    pr = refs[5 * G:6 * G]
    sem_si = refs[6 * G:7 * G]
    sem_di = refs[7 * G:8 * G]
    sem_kv = refs[8 * G:9 * G]
    sem_q = refs[9 * G:10 * G]
    sem_pe = refs[10 * G:11 * G]
    sem_eo = refs[11 * G:12 * G]
    wbuf = refs[12 * G]
    zbuf = refs[12 * G + 1]
    dsta = refs[12 * G + 2]
    wv_acc = refs[12 * G + 3]
    z_acc = refs[12 * G + 4]

    c = lax.axis_index("c")
    s = lax.axis_index("s")
    cN = c * N
    cE = c * E
    zv = jnp.zeros((16,), F32)
    lane = lax.iota(jnp.int32, 16)
    NT = (E // CH) // NS
    NG = NT // G

    @plsc.parallel_loop(0, GE, unroll=4)
    def _zk(i):
        for k in range(HALF // 16):
            wbuf[i, pl.ds(k * 16, 16)] = zv

    @plsc.parallel_loop(0, GE * 8 // 16, unroll=2)
    def _zz(i):
        j = i * 16 + lane
        plsc.store_scatter(zbuf, [j >> 3, j & 7], zv)

    for j in range(NPT // GE):
        r0 = s * NPT + j * GE
        pltpu.sync_copy(wbuf, wv_acc.at[pl.ds(r0, GE)])
        pltpu.sync_copy(zbuf, z_acc.at[pl.ds(r0, GE)])
    plsc.subcore_barrier()

    def _fire_idx(b, t):
        base = (s + NS * t) * CH
        return (pltpu.async_copy(srch.at[pl.ds(base, CH)], src_i[b], sem_si[b]),
                pltpu.async_copy(dsth.at[pl.ds(base, CH)], dst_i[b], sem_di[b]))

    def _fire_in(b, t):
        base = (s + NS * t) * CH
        src_i[b][pl.ds(0, 16)] = src_i[b][pl.ds(0, 16)] + cN
        dst_g[b][pl.ds(0, 16)] = dst_i[b][pl.ds(0, 16)] + cN
        dsta[pl.ds(b * CH, 16)] = dst_i[b][pl.ds(0, 16)]
        return (pltpu.async_copy(kvt.at[src_i[b]], kvr[b], sem_kv[b]),
                pltpu.async_copy(qt.at[dst_g[b]], qr[b], sem_q[b]),
                pltpu.async_copy(pet.at[pl.ds(cE + base, CH)], pr[b], sem_pe[b]))

    def _compute(b):
        rowz = b * CH + lane
        for hh in range(4):
            @plsc.parallel_loop(0, DH, carry=zv, unroll=8)
            def acc(d2i, a, _hh=hh, _b=b):
                col = jnp.full((16,), _hh * DH, jnp.int32) + d2i
                kv = plsc.load_gather(kvr[_b], [lane, col])
                qv = plsc.load_gather(qr[_b], [lane, col])
                pv = plsc.load_gather(pr[_b], [lane, col])
                sv = kv * qv * pv
                plsc.store_scatter(qr[_b], [lane, col], sv)
                return a + sv
            sexp = jnp.exp(jnp.clip(acc, -5.0, 5.0))
            plsc.store_scatter(zbuf, [rowz, jnp.full((16,), hh, jnp.int32)], sexp)

            @plsc.parallel_loop(0, DH, unroll=8)
            def _dims2(d2i, _hh=hh, _sexp=sexp, _b=b, _rowz=rowz):
                col = jnp.full((16,), _hh * DH, jnp.int32) + d2i
                vv = plsc.load_gather(kvr[_b], [lane, col + HALF])
                plsc.store_scatter(wbuf, [_rowz, col], vv * _sexp)

    def _group(gi, _):
        t0 = gi * G
        idx_d = []
        for b in range(G):
            idx_d.extend(_fire_idx(b, t0 + b))
        for d in idx_d:
            d.wait()
        in_d = [_fire_in(b, t0 + b) for b in range(G)]
        out_d = []
        for b in range(G):
            base = (s + NS * (t0 + b)) * CH
            for d in in_d[b]:
                d.wait()
            _compute(b)
            out_d.append(
                pltpu.async_copy(qr[b], eattn_o.at[pl.ds(cE + base, CH)],
                                 sem_eo[b]))
        pltpu.sync_copy(wbuf, wv_acc.at[dsta], add=True)
        pltpu.sync_copy(zbuf, z_acc.at[dsta], add=True)
        for d in out_d:
            d.wait()
        return 0
    lax.fori_loop(0, NG, _group, 0)

    t = NT - 1
    base = (s + NS * t) * CH
    d0, d1 = _fire_idx(0, t)
    d0.wait()
    d1.wait()
    for d in _fire_in(0, t):
        d.wait()
    _compute(0)
    pltpu.sync_copy(qr[0], eattn_o.at[pl.ds(cE + base, CH)])
    pltpu.sync_copy(wbuf.at[pl.ds(0, CH)], wv_acc.at[dst_i[0]], add=True)
    pltpu.sync_copy(zbuf.at[pl.ds(0, CH)], z_acc.at[dst_i[0]], add=True)
    plsc.subcore_barrier()

    eps = jnp.float32(1e-6)

    def _divchunk(j, _):
        r0 = s * NPT + j * CH
        pltpu.sync_copy(wv_acc.at[pl.ds(r0, CH)], qr[0])
        pltpu.sync_copy(z_acc.at[pl.ds(r0, CH)], zbuf.at[pl.ds(0, CH)])

        @plsc.parallel_loop(0, CH, unroll=4)
        def _div(r):
            for head in range(4):
                zb = plsc.load_gather(
                    zbuf, [jnp.full((16,), r, jnp.int32),
                           jnp.full((16,), head, jnp.int32)])
                rec = jnp.float32(1.0) / (zb + eps)
                for k in (2 * head, 2 * head + 1):
                    qr[0][r, pl.ds(k * 16, 16)] = qr[0][r, pl.ds(k * 16, 16)] * rec
        pltpu.sync_copy(qr[0], hattn_o.at[pl.ds(c * NPAD + r0, CH)])
        return 0
    lax.fori_loop(0, NPT // CH, _divchunk, 0)


def _sc_attention(src, dst, kvt, qt, pet):
    mesh = plsc.VectorSubcoreMesh(core_axis_name="c", subcore_axis_name="s")
    scratch = (
        [pltpu.VMEM((CH,), jnp.int32) for _ in range(G)]
        + [pltpu.VMEM((CH,), jnp.int32) for _ in range(G)]
        + [pltpu.VMEM((CH,), jnp.int32) for _ in range(G)]
        + [pltpu.VMEM((CH, 2 * HALF), F32) for _ in range(G)]
        + [pltpu.VMEM((CH, HALF), F32) for _ in range(G)]
        + [pltpu.VMEM((CH, HALF), F32) for _ in range(G)]
        + [pltpu.SemaphoreType.DMA for _ in range(6 * G)]
        + [pltpu.VMEM((GE, HALF), F32),
           pltpu.VMEM((GE, 8), F32),
           pltpu.VMEM((GE,), jnp.int32)]
        + [pltpu.VMEM_SHARED((NPAD, HALF), F32),
           pltpu.VMEM_SHARED((NPAD, 8), F32)]
    )
    kern = pl.kernel(
        _sc_attention_body,
        out_type=[
            jax.ShapeDtypeStruct((2 * E, HALF), F32),
            jax.ShapeDtypeStruct((2 * NPAD, HALF), F32),
        ],
        mesh=mesh,
        scratch_types=scratch,
        compiler_params=pltpu.CompilerParams(
            needs_layout_passes=False, use_tc_tiling_on_sc=False),
    )
    return kern(src, dst, kvt, qt, pet)


def _postA_body(x_ref, aA_ref, aB_ref, wA_ref, wB_ref, b_ref,
                out_ref, s_ref, q_ref):
    i = pl.program_id(0)
    acc = (x_ref[...]
           + jnp.dot(aA_ref[0], wA_ref[...], preferred_element_type=F32)
           + jnp.dot(aB_ref[0], wB_ref[...], preferred_element_type=F32)
           + b_ref[...])
    out_ref[...] = acc

    @pl.when(i == 0)
    def _():
        s_ref[...] = jnp.zeros_like(s_ref)
        q_ref[...] = jnp.zeros_like(q_ref)

    s_ref[...] += jnp.sum(acc, axis=0, keepdims=True)
    q_ref[...] += jnp.sum(acc * acc, axis=0, keepdims=True)


def _postA(x, attn2, wA, wB, b, bn):
    n = x.shape[0]
    sspec = pl.BlockSpec((1, D), lambda i: (0, 0))
    return pl.pallas_call(
        _postA_body,
        grid=(n // bn,),
        in_specs=[pl.BlockSpec((bn, D), lambda i: (i, 0)),
                  pl.BlockSpec((1, bn, HALF), lambda i: (0, i, 0)),
                  pl.BlockSpec((1, bn, HALF), lambda i: (1, i, 0)),
                  pl.BlockSpec((HALF, D), lambda i: (0, 0)),
                  pl.BlockSpec((HALF, D), lambda i: (0, 0)),
                  pl.BlockSpec((1, D), lambda i: (0, 0))],
        out_specs=[pl.BlockSpec((bn, D), lambda i: (i, 0)), sspec, sspec],
        out_shape=[jax.ShapeDtypeStruct((n, D), F32),
                   jax.ShapeDtypeStruct((1, D), F32),
                   jax.ShapeDtypeStruct((1, D), F32)],
    )(x, attn2, attn2, wA, wB, b)


def _postB_body(x_ref, a1_ref, c1_ref, w1_ref, b1_ref, w2_ref, b2_ref,
                out_ref, s_ref, q_ref):
    i = pl.program_id(0)
    x = x_ref[...] * a1_ref[...] + c1_ref[...]
    t = jnp.maximum(jnp.dot(x, w1_ref[...], preferred_element_type=F32)
                    + b1_ref[...], 0.0)
    y = x + jnp.dot(t, w2_ref[...], preferred_element_type=F32) + b2_ref[...]
    out_ref[...] = y

    @pl.when(i == 0)
    def _():
        s_ref[...] = jnp.zeros_like(s_ref)
        q_ref[...] = jnp.zeros_like(q_ref)

    s_ref[...] += jnp.sum(y, axis=0, keepdims=True)
    q_ref[...] += jnp.sum(y * y, axis=0, keepdims=True)


def _postB(x, a1, c1, w1, b1, w2, b2, bn):
    n = x.shape[0]
    sspec = pl.BlockSpec((1, D), lambda i: (0, 0))
    return pl.pallas_call(
        _postB_body,
        grid=(n // bn,),
        in_specs=[pl.BlockSpec((bn, D), lambda i: (i, 0)),
                  pl.BlockSpec((1, D), lambda i: (0, 0)),
                  pl.BlockSpec((1, D), lambda i: (0, 0)),
                  pl.BlockSpec((D, 2 * D), lambda i: (0, 0)),
                  pl.BlockSpec((1, 2 * D), lambda i: (0, 0)),
                  pl.BlockSpec((2 * D, D), lambda i: (0, 0)),
                  pl.BlockSpec((1, D), lambda i: (0, 0))],
        out_specs=[pl.BlockSpec((bn, D), lambda i: (i, 0)), sspec, sspec],
        out_shape=[jax.ShapeDtypeStruct((n, D), F32),
                   jax.ShapeDtypeStruct((1, D), F32),
                   jax.ShapeDtypeStruct((1, D), F32)],
    )(x, a1, c1, w1, b1, w2, b2)


def _postC_body(x_ref, a_ref, c_ref, out_ref):
    out_ref[...] = x_ref[...] * a_ref[...] + c_ref[...]


def _postC(x, a, c, bn):
    n = x.shape[0]
    return pl.pallas_call(
        _postC_body,
        grid=(n // bn,),
        in_specs=[pl.BlockSpec((bn, D), lambda i: (i, 0)),
                  pl.BlockSpec((1, D), lambda i: (0, 0)),
                  pl.BlockSpec((1, D), lambda i: (0, 0))],
        out_specs=pl.BlockSpec((bn, D), lambda i: (i, 0)),
        out_shape=jax.ShapeDtypeStruct((n, D), F32),
    )(x, a, c)


def _bn_coef(ssum, sqsum, n, g, b):
    mu = ssum[0] / n
    var = sqsum[0] / n - mu * mu
    a = g / jnp.sqrt(var + 1e-5)
    return a.reshape(1, D), (b - mu * a).reshape(1, D)


def kernel(h, e, edge_index, WQ, WK, WV, We, WOh, bOh, WOe, bOe,
           W1h, b1h, W2h, b2h, W1e, b1e, W2e, b2e,
           g1h, be1h, g1e, be1e, g2h, be2h, g2e, be2e):
    q2, kv2 = _proj3(h, WQ, WK, WV, bn=2000)
    pe2 = _proj1(e, We, bn=2000)

    eattn, hattn = _sc_attention(
        edge_index[0], edge_index[1],
        kv2.reshape(2 * N, 2 * HALF), q2.reshape(2 * N, HALF),
        pe2.reshape(2 * E, HALF))

    hattn2 = hattn.reshape(2, NPAD, HALF)
    eattn2 = eattn.reshape(2, E, HALF)

    hh, hs1, hq1 = _postA(h, hattn2, WOh[:HALF], WOh[HALF:],
                          bOh.reshape(1, D), bn=2000)
    ee, es1, eq1 = _postA(e, eattn2, WOe[:HALF], WOe[HALF:],
                          bOe.reshape(1, D), bn=2000)

    ha1, hc1 = _bn_coef(hs1, hq1, N, g1h, be1h)
    ea1, ec1 = _bn_coef(es1, eq1, E, g1e, be1e)

    hy, hs2, hq2 = _postB(hh, ha1, hc1, W1h, b1h.reshape(1, 2 * D),
                          W2h, b2h.reshape(1, D), bn=2000)
    ey, es2, eq2 = _postB(ee, ea1, ec1, W1e, b1e.reshape(1, 2 * D),
                          W2e, b2e.reshape(1, D), bn=2000)

    ha2, hc2 = _bn_coef(hs2, hq2, N, g2h, be2h)
    ea2, ec2 = _bn_coef(es2, eq2, E, g2e, be2e)

    h_out = _postC(hy, ha2, hc2, bn=2000)
    e_out = _postC(ey, ea2, ec2, bn=2000)
    return (h_out, e_out)


# speedup vs baseline: 12.9443x; 2.2617x over previous
import functools

import jax
import jax.numpy as jnp
from jax import lax
from jax.experimental import pallas as pl
from jax.experimental.pallas import tpu as pltpu
from jax.experimental.pallas import tpu_sc as plsc

N = 10000
E = 160000
D = 256
H = 8
DH = 32
HALF = 128
NS = 16
EPT = E // NS
CH = 16
NPAD = 10240
NPT = NPAD // NS
F32 = jnp.float32


def _proj3_body(x_ref, wq_ref, wk_ref, wv_ref, q_ref, kv_ref):
    x = x_ref[...]
    scale = jnp.float32(1.0 / (DH ** 0.5))
    q_ref[0] = jnp.dot(x, wq_ref[...], preferred_element_type=F32) * scale
    kv_ref[0] = jnp.concatenate(
        [jnp.dot(x, wk_ref[...], preferred_element_type=F32),
         jnp.dot(x, wv_ref[...], preferred_element_type=F32)], axis=1)


def _proj3(x, wq, wk, wv, bn):
    n = x.shape[0]
    wspec = pl.BlockSpec((D, HALF), lambda c, i: (0, c))
    return pl.pallas_call(
        _proj3_body,
        grid=(2, n // bn),
        in_specs=[pl.BlockSpec((bn, D), lambda c, i: (i, 0)), wspec, wspec, wspec],
        out_specs=[pl.BlockSpec((1, bn, HALF), lambda c, i: (c, i, 0)),
                   pl.BlockSpec((1, bn, 2 * HALF), lambda c, i: (c, i, 0))],
        out_shape=[jax.ShapeDtypeStruct((2, n, HALF), F32),
                   jax.ShapeDtypeStruct((2, n, 2 * HALF), F32)],
    )(x, wq, wk, wv)


def _proj1_body(x_ref, w_ref, o_ref):
    o_ref[0] = jnp.dot(x_ref[...], w_ref[...], preferred_element_type=F32)


def _proj1(x, w, bn):
    n = x.shape[0]
    return pl.pallas_call(
        _proj1_body,
        grid=(2, n // bn),
        in_specs=[pl.BlockSpec((bn, D), lambda c, i: (i, 0)),
                  pl.BlockSpec((D, HALF), lambda c, i: (0, c))],
        out_specs=pl.BlockSpec((1, bn, HALF), lambda c, i: (c, i, 0)),
        out_shape=jax.ShapeDtypeStruct((2, n, HALF), F32),
    )(x, w)


G = 4
GE = G * CH


def _sc_attention_body(srch, dsth, kvt, qt, pet, eattn_o, hattn_o,
                       *refs):
    src_i = refs[0:G]
    dst_i = refs[G:2 * G]
    dst_g = refs[2 * G:3 * G]
    kvr = refs[3 * G:4 * G]
    qr = refs[4 * G:5 * G]
    pr = refs[5 * G:6 * G]
    sem_si = refs[6 * G:7 * G]
    sem_di = refs[7 * G:8 * G]
    sem_kv = refs[8 * G:9 * G]
    sem_q = refs[9 * G:10 * G]
    sem_pe = refs[10 * G:11 * G]
    sem_eo = refs[11 * G:12 * G]
    wbuf = refs[12 * G]
    zbuf = refs[12 * G + 1]
    dsta = refs[12 * G + 2]
    wv_acc = refs[12 * G + 3]
    z_acc = refs[12 * G + 4]

    c = lax.axis_index("c")
    s = lax.axis_index("s")
    cN = c * N
    cE = c * E
    zv = jnp.zeros((16,), F32)
    lane = lax.iota(jnp.int32, 16)
    NT = (E // CH) // NS
    NG = NT // G

    @plsc.parallel_loop(0, GE, unroll=4)
    def _zk(i):
        for k in range(HALF // 16):
            wbuf[i, pl.ds(k * 16, 16)] = zv

    @plsc.parallel_loop(0, GE * 8 // 16, unroll=2)
    def _zz(i):
        j = i * 16 + lane
        plsc.store_scatter(zbuf, [j >> 3, j & 7], zv)

    for j in range(NPT // GE):
        r0 = s * NPT + j * GE
        pltpu.sync_copy(wbuf, wv_acc.at[pl.ds(r0, GE)])
        pltpu.sync_copy(zbuf, z_acc.at[pl.ds(r0, GE)])
    plsc.subcore_barrier()

    def _fire_idx(b, t):
        base = (s + NS * t) * CH
        return (pltpu.async_copy(srch.at[pl.ds(base, CH)], src_i[b], sem_si[b]),
                pltpu.async_copy(dsth.at[pl.ds(base, CH)], dst_i[b], sem_di[b]))

    def _fire_in(b, t):
        base = (s + NS * t) * CH
        src_i[b][pl.ds(0, 16)] = src_i[b][pl.ds(0, 16)] + cN
        dst_g[b][pl.ds(0, 16)] = dst_i[b][pl.ds(0, 16)] + cN
        dsta[pl.ds(b * CH, 16)] = dst_i[b][pl.ds(0, 16)]
        return (pltpu.async_copy(kvt.at[src_i[b]], kvr[b], sem_kv[b]),
                pltpu.async_copy(qt.at[dst_g[b]], qr[b], sem_q[b]),
                pltpu.async_copy(pet.at[pl.ds(cE + base, CH)], pr[b], sem_pe[b]))

    def _compute(b):
        @plsc.parallel_loop(0, CH, unroll=2)
        def _edges(e, _b=b):
            rowz = _b * CH + e
            zrow = zv
            for hh in range(4):
                c0 = hh * DH
                c1 = c0 + 16
                k0 = kvr[_b][e, pl.ds(c0, 16)]
                k1 = kvr[_b][e, pl.ds(c1, 16)]
                q0 = qr[_b][e, pl.ds(c0, 16)]
                q1 = qr[_b][e, pl.ds(c1, 16)]
                p0 = pr[_b][e, pl.ds(c0, 16)]
                p1 = pr[_b][e, pl.ds(c1, 16)]
                s0 = k0 * q0 * p0
                s1 = k1 * q1 * p1
                qr[_b][e, pl.ds(c0, 16)] = s0
                qr[_b][e, pl.ds(c1, 16)] = s1
                tot = jnp.sum(s0 + s1)
                se = jnp.exp(jnp.clip(jnp.full((16,), tot, F32), -5.0, 5.0))
                zrow = jnp.where(lane == hh, se, zrow)
                v0 = kvr[_b][e, pl.ds(c0 + HALF, 16)]
                v1 = kvr[_b][e, pl.ds(c1 + HALF, 16)]
                wbuf[rowz, pl.ds(c0, 16)] = v0 * se
                wbuf[rowz, pl.ds(c1, 16)] = v1 * se
            plsc.store_scatter(
                zbuf, [jnp.full((16,), rowz, jnp.int32), lane], zrow,
                mask=lane < 8)

    def _group(gi, _):
        t0 = gi * G
        idx_d = []
        for b in range(G):
            idx_d.extend(_fire_idx(b, t0 + b))
        for d in idx_d:
            d.wait()
        in_d = [_fire_in(b, t0 + b) for b in range(G)]
        out_d = []
        for b in range(G):
            base = (s + NS * (t0 + b)) * CH
            for d in in_d[b]:
                d.wait()
            _compute(b)
            out_d.append(
                pltpu.async_copy(qr[b], eattn_o.at[pl.ds(cE + base, CH)],
                                 sem_eo[b]))
        pltpu.sync_copy(wbuf, wv_acc.at[dsta], add=True)
        pltpu.sync_copy(zbuf, z_acc.at[dsta], add=True)
        for d in out_d:
            d.wait()
        return 0
    lax.fori_loop(0, NG, _group, 0)

    t = NT - 1
    base = (s + NS * t) * CH
    d0, d1 = _fire_idx(0, t)
    d0.wait()
    d1.wait()
    for d in _fire_in(0, t):
        d.wait()
    _compute(0)
    pltpu.sync_copy(qr[0], eattn_o.at[pl.ds(cE + base, CH)])
    pltpu.sync_copy(wbuf.at[pl.ds(0, CH)], wv_acc.at[dst_i[0]], add=True)
    pltpu.sync_copy(zbuf.at[pl.ds(0, CH)], z_acc.at[dst_i[0]], add=True)
    plsc.subcore_barrier()

    eps = jnp.float32(1e-6)

    def _divchunk(j, _):
        r0 = s * NPT + j * CH
        pltpu.sync_copy(wv_acc.at[pl.ds(r0, CH)], qr[0])
        pltpu.sync_copy(z_acc.at[pl.ds(r0, CH)], zbuf.at[pl.ds(0, CH)])

        @plsc.parallel_loop(0, CH, unroll=4)
        def _div(r):
            for head in range(4):
                zb = plsc.load_gather(
                    zbuf, [jnp.full((16,), r, jnp.int32),
                           jnp.full((16,), head, jnp.int32)])
                rec = jnp.float32(1.0) / (zb + eps)
                for k in (2 * head, 2 * head + 1):
                    qr[0][r, pl.ds(k * 16, 16)] = qr[0][r, pl.ds(k * 16, 16)] * rec
        pltpu.sync_copy(qr[0], hattn_o.at[pl.ds(c * NPAD + r0, CH)])
        return 0
    lax.fori_loop(0, NPT // CH, _divchunk, 0)


def _sc_attention(src, dst, kvt, qt, pet):
    mesh = plsc.VectorSubcoreMesh(core_axis_name="c", subcore_axis_name="s")
    scratch = (
        [pltpu.VMEM((CH,), jnp.int32) for _ in range(G)]
        + [pltpu.VMEM((CH,), jnp.int32) for _ in range(G)]
        + [pltpu.VMEM((CH,), jnp.int32) for _ in range(G)]
        + [pltpu.VMEM((CH, 2 * HALF), F32) for _ in range(G)]
        + [pltpu.VMEM((CH, HALF), F32) for _ in range(G)]
        + [pltpu.VMEM((CH, HALF), F32) for _ in range(G)]
        + [pltpu.SemaphoreType.DMA for _ in range(6 * G)]
        + [pltpu.VMEM((GE, HALF), F32),
           pltpu.VMEM((GE, 8), F32),
           pltpu.VMEM((GE,), jnp.int32)]
        + [pltpu.VMEM_SHARED((NPAD, HALF), F32),
           pltpu.VMEM_SHARED((NPAD, 8), F32)]
    )
    kern = pl.kernel(
        _sc_attention_body,
        out_type=[
            jax.ShapeDtypeStruct((2 * E, HALF), F32),
            jax.ShapeDtypeStruct((2 * NPAD, HALF), F32),
        ],
        mesh=mesh,
        scratch_types=scratch,
        compiler_params=pltpu.CompilerParams(
            needs_layout_passes=False, use_tc_tiling_on_sc=False),
    )
    return kern(src, dst, kvt, qt, pet)


def _postA_body(x_ref, aA_ref, aB_ref, wA_ref, wB_ref, b_ref,
                out_ref, s_ref, q_ref):
    i = pl.program_id(0)
    acc = (x_ref[...]
           + jnp.dot(aA_ref[0], wA_ref[...], preferred_element_type=F32)
           + jnp.dot(aB_ref[0], wB_ref[...], preferred_element_type=F32)
           + b_ref[...])
    out_ref[...] = acc

    @pl.when(i == 0)
    def _():
        s_ref[...] = jnp.zeros_like(s_ref)
        q_ref[...] = jnp.zeros_like(q_ref)

    s_ref[...] += jnp.sum(acc, axis=0, keepdims=True)
    q_ref[...] += jnp.sum(acc * acc, axis=0, keepdims=True)


def _postA(x, attn2, wA, wB, b, bn):
    n = x.shape[0]
    sspec = pl.BlockSpec((1, D), lambda i: (0, 0))
    return pl.pallas_call(
        _postA_body,
        grid=(n // bn,),
        in_specs=[pl.BlockSpec((bn, D), lambda i: (i, 0)),
                  pl.BlockSpec((1, bn, HALF), lambda i: (0, i, 0)),
                  pl.BlockSpec((1, bn, HALF), lambda i: (1, i, 0)),
                  pl.BlockSpec((HALF, D), lambda i: (0, 0)),
                  pl.BlockSpec((HALF, D), lambda i: (0, 0)),
                  pl.BlockSpec((1, D), lambda i: (0, 0))],
        out_specs=[pl.BlockSpec((bn, D), lambda i: (i, 0)), sspec, sspec],
        out_shape=[jax.ShapeDtypeStruct((n, D), F32),
                   jax.ShapeDtypeStruct((1, D), F32),
                   jax.ShapeDtypeStruct((1, D), F32)],
    )(x, attn2, attn2, wA, wB, b)


def _postB_body(x_ref, a1_ref, c1_ref, w1_ref, b1_ref, w2_ref, b2_ref,
                out_ref, s_ref, q_ref):
    i = pl.program_id(0)
    x = x_ref[...] * a1_ref[...] + c1_ref[...]
    t = jnp.maximum(jnp.dot(x, w1_ref[...], preferred_element_type=F32)
                    + b1_ref[...], 0.0)
    y = x + jnp.dot(t, w2_ref[...], preferred_element_type=F32) + b2_ref[...]
    out_ref[...] = y

    @pl.when(i == 0)
    def _():
        s_ref[...] = jnp.zeros_like(s_ref)
        q_ref[...] = jnp.zeros_like(q_ref)

    s_ref[...] += jnp.sum(y, axis=0, keepdims=True)
    q_ref[...] += jnp.sum(y * y, axis=0, keepdims=True)


def _postB(x, a1, c1, w1, b1, w2, b2, bn):
    n = x.shape[0]
    sspec = pl.BlockSpec((1, D), lambda i: (0, 0))
    return pl.pallas_call(
        _postB_body,
        grid=(n // bn,),
        in_specs=[pl.BlockSpec((bn, D), lambda i: (i, 0)),
                  pl.BlockSpec((1, D), lambda i: (0, 0)),
                  pl.BlockSpec((1, D), lambda i: (0, 0)),
                  pl.BlockSpec((D, 2 * D), lambda i: (0, 0)),
                  pl.BlockSpec((1, 2 * D), lambda i: (0, 0)),
                  pl.BlockSpec((2 * D, D), lambda i: (0, 0)),
                  pl.BlockSpec((1, D), lambda i: (0, 0))],
        out_specs=[pl.BlockSpec((bn, D), lambda i: (i, 0)), sspec, sspec],
        out_shape=[jax.ShapeDtypeStruct((n, D), F32),
                   jax.ShapeDtypeStruct((1, D), F32),
                   jax.ShapeDtypeStruct((1, D), F32)],
    )(x, a1, c1, w1, b1, w2, b2)


def _postC_body(x_ref, a_ref, c_ref, out_ref):
    out_ref[...] = x_ref[...] * a_ref[...] + c_ref[...]


def _postC(x, a, c, bn):
    n = x.shape[0]
    return pl.pallas_call(
        _postC_body,
        grid=(n // bn,),
        in_specs=[pl.BlockSpec((bn, D), lambda i: (i, 0)),
                  pl.BlockSpec((1, D), lambda i: (0, 0)),
                  pl.BlockSpec((1, D), lambda i: (0, 0))],
        out_specs=pl.BlockSpec((bn, D), lambda i: (i, 0)),
        out_shape=jax.ShapeDtypeStruct((n, D), F32),
    )(x, a, c)


def _bn_coef(ssum, sqsum, n, g, b):
    mu = ssum[0] / n
    var = sqsum[0] / n - mu * mu
    a = g / jnp.sqrt(var + 1e-5)
    return a.reshape(1, D), (b - mu * a).reshape(1, D)


def kernel(h, e, edge_index, WQ, WK, WV, We, WOh, bOh, WOe, bOe,
           W1h, b1h, W2h, b2h, W1e, b1e, W2e, b2e,
           g1h, be1h, g1e, be1e, g2h, be2h, g2e, be2e):
    q2, kv2 = _proj3(h, WQ, WK, WV, bn=2000)
    pe2 = _proj1(e, We, bn=2000)

    eattn, hattn = _sc_attention(
        edge_index[0], edge_index[1],
        kv2.reshape(2 * N, 2 * HALF), q2.reshape(2 * N, HALF),
        pe2.reshape(2 * E, HALF))

    hattn2 = hattn.reshape(2, NPAD, HALF)
    eattn2 = eattn.reshape(2, E, HALF)

    hh, hs1, hq1 = _postA(h, hattn2, WOh[:HALF], WOh[HALF:],
                          bOh.reshape(1, D), bn=2000)
    ee, es1, eq1 = _postA(e, eattn2, WOe[:HALF], WOe[HALF:],
                          bOe.reshape(1, D), bn=2000)

    ha1, hc1 = _bn_coef(hs1, hq1, N, g1h, be1h)
    ea1, ec1 = _bn_coef(es1, eq1, E, g1e, be1e)

    hy, hs2, hq2 = _postB(hh, ha1, hc1, W1h, b1h.reshape(1, 2 * D),
                          W2h, b2h.reshape(1, D), bn=2000)
    ey, es2, eq2 = _postB(ee, ea1, ec1, W1e, b1e.reshape(1, 2 * D),
                          W2e, b2e.reshape(1, D), bn=2000)

    ha2, hc2 = _bn_coef(hs2, hq2, N, g2h, be2h)
    ea2, ec2 = _bn_coef(es2, eq2, E, g2e, be2e)

    h_out = _postC(hy, ha2, hc2, bn=2000)
    e_out = _postC(ey, ea2, ec2, bn=2000)
    return (h_out, e_out)
